# baseline (speedup 1.0000x reference)
import functools

import jax
import jax.numpy as jnp
from jax import lax
from jax.experimental import pallas as pl
from jax.experimental.pallas import tpu as pltpu
from jax.experimental.pallas import tpu_sc as plsc

NC = 2
NS = 16
NW = NC * NS

D = 64
CH = 128


NBUF = 4


def _make_gather(batch, hist):
    rows_w = batch // NW
    ngrp = rows_w // NBUF
    assert batch % NW == 0 and rows_w % NBUF == 0
    mesh = plsc.VectorSubcoreMesh(core_axis_name="c", subcore_axis_name="s")

    @functools.partial(
        pl.kernel,
        mesh=mesh,
        compiler_params=pltpu.CompilerParams(use_tc_tiling_on_sc=False),
        out_type=jax.ShapeDtypeStruct((batch, hist, D), jnp.float32),
        scratch_types=[
            pltpu.VMEM((rows_w, hist), jnp.int32),
            pltpu.VMEM((NBUF, hist, D), jnp.float32),
            pltpu.SemaphoreType.DMA((NBUF,)),
        ],
    )
    def gather(idx_hbm, table_hbm, out_hbm, idx_v, rows_v, sem):
        wid = lax.axis_index("s") * NC + lax.axis_index("c")
        base = wid * rows_w
        pltpu.sync_copy(idx_hbm.at[pl.ds(base, rows_w)], idx_v)

        def start(c, b):
            pltpu.async_copy(table_hbm.at[idx_v.at[c]], rows_v.at[b], sem.at[b])

        def finish(c, b):
            pltpu.make_async_copy(
                table_hbm.at[idx_v.at[c]], rows_v.at[b], sem.at[b]
            ).wait()
            pltpu.sync_copy(rows_v.at[b], out_hbm.at[base + c])

        for b in range(NBUF):
            start(b, b)

        def group(g, carry):
            for b in range(NBUF):
                c = g * NBUF + b
                finish(c, b)
                start(c + NBUF, b)
            return carry

        lax.fori_loop(0, ngrp - 1, group, 0)

        for b in range(NBUF):
            finish((ngrp - 1) * NBUF + b, b)

    return gather


def kernel(indices, weight):
    batch, hist = indices.shape
    return _make_gather(batch, hist)(indices, weight)
